# baseline (speedup 1.0000x reference)
import functools

import jax
import jax.numpy as jnp
from jax import lax
from jax.experimental import pallas as pl
from jax.experimental.pallas import tpu as pltpu
from jax.experimental.pallas import tpu_sc as plsc

N = 10000
E = 320000
D = 128
BN_EPS = 1e-5

NC = 2
NS = 16
NW = NC * NS
EPW = E // NW
C = 80
CHUNKS = EPW // C
RPS = 624
TAIL = N - NS * RPS


def _sc_aggregate(x, src, dst, zeros):
    mesh = plsc.VectorSubcoreMesh(core_axis_name="c", subcore_axis_name="s")

    @functools.partial(
        pl.kernel,
        mesh=mesh,
        out_type=jax.ShapeDtypeStruct((NC, N, D), jnp.float32),
        scratch_types=[
            pltpu.VMEM((EPW,), jnp.int32),
            pltpu.VMEM((CHUNKS, C), jnp.int32),
            pltpu.VMEM((2, C, D), jnp.float32),
            pltpu.VMEM_SHARED((N, D), jnp.float32),
            pltpu.SemaphoreType.DMA,
            pltpu.SemaphoreType.DMA,
            pltpu.SemaphoreType.DMA,
            pltpu.SemaphoreType.DMA,
        ],
    )
    def sc_agg(x_hbm, src_hbm, dst_hbm, zeros_hbm, out_hbm,
               src_v, dst_v, rows_v, agg_sh, sem0, sem1, ssem0, ssem1):
        cid = lax.axis_index("c")
        sid = lax.axis_index("s")
        wid = sid * NC + cid

        pltpu.sync_copy(zeros_hbm, agg_sh.at[pl.ds(sid * RPS, RPS)])

        @pl.when(sid == NS - 1)
        def _():
            pltpu.sync_copy(zeros_hbm.at[pl.ds(0, TAIL)],
                            agg_sh.at[pl.ds(NS * RPS, TAIL)])

        pltpu.sync_copy(src_hbm.at[wid], src_v)
        pltpu.sync_copy(dst_hbm.at[wid], dst_v)
        plsc.subcore_barrier()

        def gather(j, buf, sem):
            pltpu.async_copy(
                x_hbm.at[src_v.at[pl.ds(j * C, C)]], rows_v.at[buf], sem)

        def wait_rows(buf, sem):
            pltpu.make_async_copy(
                x_hbm.at[pl.ds(0, C)], rows_v.at[buf], sem).wait()

        def scatter_add(j, buf, sem):
            pltpu.async_copy(rows_v.at[buf], agg_sh.at[dst_v.at[j]], sem,
                             add=True)

        gather(0, 0, sem0)
        gather(1, 1, sem1)

        @pl.loop(0, CHUNKS - 3, step=2)
        def _(j):
            wait_rows(0, sem0)
            scatter_add(j, 0, ssem0)
            wait_rows(1, sem1)
            scatter_add(j + 1, 1, ssem1)
            wait_rows(0, ssem0)
            gather(j + 2, 0, sem0)
            wait_rows(1, ssem1)
            gather(j + 3, 1, sem1)

        wait_rows(0, sem0)
        scatter_add(CHUNKS - 3, 0, ssem0)
        wait_rows(0, ssem0)
        gather(CHUNKS - 1, 0, sem0)
        wait_rows(1, sem1)
        scatter_add(CHUNKS - 2, 1, ssem1)
        wait_rows(0, sem0)
        scatter_add(CHUNKS - 1, 0, ssem0)
        wait_rows(1, ssem1)
        wait_rows(0, ssem0)

        plsc.subcore_barrier()
        pltpu.sync_copy(agg_sh.at[pl.ds(sid * RPS, RPS)],
                        out_hbm.at[cid, pl.ds(sid * RPS, RPS)])

        @pl.when(sid == NS - 1)
        def _():
            pltpu.sync_copy(agg_sh.at[pl.ds(NS * RPS, TAIL)],
                            out_hbm.at[cid, pl.ds(NS * RPS, TAIL)])

    return sc_agg(x, src, dst, zeros)


def _mlp_body(x_ref, aggp_ref, w1_ref, b1_ref, gamma_ref, beta_ref,
              w2_ref, b2_ref, eps_ref, y_ref):
    agg = aggp_ref[0] + aggp_ref[1]
    out = (1.0 + eps_ref[0, 0]) * x_ref[...] + agg
    h = lax.dot_general(out, w1_ref[...], (((1,), (1,)), ((), ())),
                        preferred_element_type=jnp.float32,
                        precision=lax.Precision.DEFAULT) + b1_ref[...]
    mu = jnp.mean(h, axis=0, keepdims=True)
    var = jnp.mean((h - mu) ** 2, axis=0, keepdims=True)
    hn = (h - mu) * (gamma_ref[...] / jnp.sqrt(var + BN_EPS)) + beta_ref[...]
    hr = jnp.maximum(hn, 0.0)
    y_ref[...] = lax.dot_general(hr, w2_ref[...], (((1,), (1,)), ((), ())),
                                 preferred_element_type=jnp.float32,
                                 precision=lax.Precision.DEFAULT) + b2_ref[...]


def kernel(x, edge_index, W1, b1, gamma, beta, W2, b2, eps):
    ei = edge_index.astype(jnp.int32)
    src = ei[0].reshape(NW, EPW)
    dst = ei[1].reshape(NW, CHUNKS, C)
    zeros = jnp.zeros((RPS, D), jnp.float32)

    aggp = _sc_aggregate(x, src, dst, zeros)

    y = pl.pallas_call(
        _mlp_body,
        out_shape=jax.ShapeDtypeStruct((N, D), jnp.float32),
    )(x, aggp,
      W1, b1.reshape(1, D), gamma.reshape(1, D), beta.reshape(1, D),
      W2, b2.reshape(1, D), eps.reshape(1, 1).astype(jnp.float32))
    return y


# speedup vs baseline: 1.2154x; 1.2154x over previous
import functools

import jax
import jax.numpy as jnp
from jax import lax
from jax.experimental import pallas as pl
from jax.experimental.pallas import tpu as pltpu
from jax.experimental.pallas import tpu_sc as plsc

N = 10000
E = 320000
D = 128
BN_EPS = 1e-5

NC = 2
NS = 16
NW = NC * NS
EPW = E // NW
C = 80
CHUNKS = EPW // C
RPS = 624
TAIL = N - NS * RPS


def _sc_aggregate(x, src, dst, zeros):
    mesh = plsc.VectorSubcoreMesh(core_axis_name="c", subcore_axis_name="s")

    @functools.partial(
        pl.kernel,
        mesh=mesh,
        out_type=jax.ShapeDtypeStruct((NC, N, D), jnp.float32),
        scratch_types=[
            pltpu.VMEM((EPW,), jnp.int32),
            pltpu.VMEM((CHUNKS, C), jnp.int32),
            pltpu.VMEM((2, C, D), jnp.float32),
            pltpu.VMEM_SHARED((N, D), jnp.float32),
            pltpu.SemaphoreType.DMA,
            pltpu.SemaphoreType.DMA,
        ],
    )
    def sc_agg(x_hbm, src_hbm, dst_hbm, zeros_hbm, out_hbm,
               src_v, dst_v, rows_v, agg_sh, sem0, sem1):
        cid = lax.axis_index("c")
        sid = lax.axis_index("s")
        wid = sid * NC + cid

        pltpu.sync_copy(zeros_hbm, agg_sh.at[pl.ds(sid * RPS, RPS)])

        @pl.when(sid == NS - 1)
        def _():
            pltpu.sync_copy(zeros_hbm.at[pl.ds(0, TAIL)],
                            agg_sh.at[pl.ds(NS * RPS, TAIL)])

        pltpu.sync_copy(src_hbm.at[wid], src_v)
        pltpu.sync_copy(dst_hbm.at[wid], dst_v)
        plsc.subcore_barrier()

        def gather(j, buf, sem):
            pltpu.async_copy(
                x_hbm.at[src_v.at[pl.ds(j * C, C)]], rows_v.at[buf], sem)

        def wait_rows(buf, sem):
            pltpu.make_async_copy(
                x_hbm.at[pl.ds(0, C)], rows_v.at[buf], sem).wait()

        def scatter_add(j, buf):
            pltpu.sync_copy(rows_v.at[buf], agg_sh.at[dst_v.at[j]], add=True)

        gather(0, 0, sem0)
        gather(1, 1, sem1)

        @pl.loop(0, CHUNKS - 3, step=2)
        def _(j):
            wait_rows(0, sem0)
            scatter_add(j, 0)
            gather(j + 2, 0, sem0)
            wait_rows(1, sem1)
            scatter_add(j + 1, 1)
            gather(j + 3, 1, sem1)

        wait_rows(0, sem0)
        scatter_add(CHUNKS - 3, 0)
        gather(CHUNKS - 1, 0, sem0)
        wait_rows(1, sem1)
        scatter_add(CHUNKS - 2, 1)
        wait_rows(0, sem0)
        scatter_add(CHUNKS - 1, 0)

        plsc.subcore_barrier()
        pltpu.sync_copy(agg_sh.at[pl.ds(sid * RPS, RPS)],
                        out_hbm.at[cid, pl.ds(sid * RPS, RPS)])

        @pl.when(sid == NS - 1)
        def _():
            pltpu.sync_copy(agg_sh.at[pl.ds(NS * RPS, TAIL)],
                            out_hbm.at[cid, pl.ds(NS * RPS, TAIL)])

    return sc_agg(x, src, dst, zeros)


def _mlp_body(x_ref, aggp_ref, w1_ref, b1_ref, gamma_ref, beta_ref,
              w2_ref, b2_ref, eps_ref, y_ref):
    agg = aggp_ref[0] + aggp_ref[1]
    out = (1.0 + eps_ref[0, 0]) * x_ref[...] + agg
    h = lax.dot_general(out, w1_ref[...], (((1,), (1,)), ((), ())),
                        preferred_element_type=jnp.float32,
                        precision=lax.Precision.DEFAULT) + b1_ref[...]
    mu = jnp.mean(h, axis=0, keepdims=True)
    var = jnp.mean((h - mu) ** 2, axis=0, keepdims=True)
    hn = (h - mu) * (gamma_ref[...] / jnp.sqrt(var + BN_EPS)) + beta_ref[...]
    hr = jnp.maximum(hn, 0.0)
    y_ref[...] = lax.dot_general(hr, w2_ref[...], (((1,), (1,)), ((), ())),
                                 preferred_element_type=jnp.float32,
                                 precision=lax.Precision.DEFAULT) + b2_ref[...]


def kernel(x, edge_index, W1, b1, gamma, beta, W2, b2, eps):
    ei = edge_index.astype(jnp.int32)
    src = ei[0].reshape(NW, EPW)
    dst = ei[1].reshape(NW, CHUNKS, C)
    zeros = jnp.zeros((RPS, D), jnp.float32)

    aggp = _sc_aggregate(x, src, dst, zeros)

    y = pl.pallas_call(
        _mlp_body,
        out_shape=jax.ShapeDtypeStruct((N, D), jnp.float32),
    )(x, aggp,
      W1, b1.reshape(1, D), gamma.reshape(1, D), beta.reshape(1, D),
      W2, b2.reshape(1, D), eps.reshape(1, 1).astype(jnp.float32))
    return y
